# baseline (speedup 1.0000x reference)
import functools

import jax
import jax.numpy as jnp
from jax import lax
from jax.experimental import pallas as pl
from jax.experimental.pallas import tpu as pltpu
from jax.experimental.pallas import tpu_sc as plsc

N = 10000
NP = 10240
E = 160000
EP = 163840
DIN = 256
H = 512
NL = 12
PAD_NODE = 10200

K = 128
EPW = EP // 16
NCH = EPW // K
ROWS_T = NP // 16

KD = 128
EPW2 = EP // 32
NCHD = EPW2 // KD
W = 4
NWIN = NCH // W

R = 1024
NB = NP // R

_mesh = plsc.VectorSubcoreMesh(core_axis_name="c", subcore_axis_name="s")


@functools.partial(
    pl.kernel,
    out_type=jax.ShapeDtypeStruct((2, 2, NP), jnp.float32),
    mesh=_mesh,
    scratch_types=[
        pltpu.VMEM_SHARED((NP,), jnp.float32),
        pltpu.VMEM_SHARED((NP,), jnp.float32),
        pltpu.VMEM((NCHD, KD), jnp.int32),
        pltpu.VMEM((NCHD, KD), jnp.int32),
        pltpu.VMEM((KD,), jnp.float32),
        pltpu.VMEM((ROWS_T,), jnp.float32),
    ],
)
def _sc_degrees(src_hbm, dst_hbm, ones_hbm, zrow_hbm, out_hbm,
                acc_s, acc_d, sidx, didx, ones_v, zrow_v):
    cid = lax.axis_index("c")
    sid = lax.axis_index("s")
    wid = sid * 2 + cid
    pltpu.sync_copy(ones_hbm, ones_v)
    pltpu.sync_copy(zrow_hbm, zrow_v)
    pltpu.sync_copy(src_hbm.at[pl.ds(wid * NCHD, NCHD)], sidx)
    pltpu.sync_copy(dst_hbm.at[pl.ds(wid * NCHD, NCHD)], didx)
    pltpu.sync_copy(zrow_v, acc_s.at[pl.ds(sid * ROWS_T, ROWS_T)])
    pltpu.sync_copy(zrow_v, acc_d.at[pl.ds(sid * ROWS_T, ROWS_T)])
    plsc.subcore_barrier()

    def body(j, carry):
        pltpu.sync_copy(ones_v, acc_s.at[sidx.at[j]], add=True)
        pltpu.sync_copy(ones_v, acc_d.at[didx.at[j]], add=True)
        return carry

    lax.fori_loop(0, NCHD, body, 0)
    plsc.subcore_barrier()
    pltpu.sync_copy(acc_s.at[pl.ds(sid * ROWS_T, ROWS_T)],
                    out_hbm.at[cid, 0, pl.ds(sid * ROWS_T, ROWS_T)])
    pltpu.sync_copy(acc_d.at[pl.ds(sid * ROWS_T, ROWS_T)],
                    out_hbm.at[cid, 1, pl.ds(sid * ROWS_T, ROWS_T)])


@functools.partial(
    pl.kernel,
    out_type=jax.ShapeDtypeStruct((4, NP, 128), jnp.float32),
    mesh=_mesh,
    scratch_types=[
        pltpu.VMEM_SHARED((NP, 128), jnp.float32),
        pltpu.VMEM((2, W, K), jnp.int32),
        pltpu.VMEM((2, W, K), jnp.int32),
        pltpu.VMEM((K, 128), jnp.float32),
        pltpu.VMEM((K, 128), jnp.float32),
        pltpu.SemaphoreType.DMA,
        pltpu.SemaphoreType.DMA,
        pltpu.SemaphoreType.DMA,
        pltpu.SemaphoreType.DMA,
        pltpu.SemaphoreType.DMA,
    ],
)
def _sc_aggregate(y_hbm, srcr_hbm, dstr_hbm, z_hbm, out_hbm,
                  acc, sidxw, didxw, buf0, buf1, sg0, sg1, ss0, ss1, si):
    cid = lax.axis_index("c")
    sid = lax.axis_index("s")
    base = sid * ROWS_T
    bufs = (buf0, buf1)
    sgs = (sg0, sg1)
    sss = (ss0, ss1)
    for half in range(2):
        cc = cid * 2 + half
        pltpu.sync_copy(z_hbm, acc.at[pl.ds(base, ROWS_T)])
        pltpu.sync_copy(srcr_hbm.at[pl.ds(sid * NCH, W)], sidxw.at[0])
        pltpu.sync_copy(dstr_hbm.at[pl.ds(sid * NCH, W)], didxw.at[0])
        plsc.subcore_barrier()

        def gather(wp, ci, buf, sem):
            pltpu.async_copy(
                y_hbm.at[cc].at[sidxw.at[wp, ci, pl.ds(0, K // 2)]],
                buf.at[pl.ds(0, K // 2)], sem)
            pltpu.async_copy(
                y_hbm.at[cc].at[sidxw.at[wp, ci, pl.ds(K // 2, K // 2)]],
                buf.at[pl.ds(K // 2, K // 2)], sem)

        def scat(idx, buf, sem):
            return pltpu.async_copy(buf, acc.at[idx], sem, add=True)

        def wait_g(buf, sem):
            pltpu.make_async_copy(y_hbm.at[cc].at[sidxw.at[0, 0]], buf,
                                  sem).wait()

        def wait_s(buf, sem):
            pltpu.make_async_copy(buf, acc.at[didxw.at[0, 0]], sem).wait()

        gather(0, 0, buf0, sg0)
        gather(0, 1, buf1, sg1)

        def wbody(w, carry):
            wpar = w % 2
            npar = 1 - wpar

            @pl.when(w < NWIN - 1)
            def _():
                pltpu.async_copy(
                    srcr_hbm.at[pl.ds(sid * NCH + (w + 1) * W, W)],
                    sidxw.at[npar], si)
                pltpu.async_copy(
                    dstr_hbm.at[pl.ds(sid * NCH + (w + 1) * W, W)],
                    didxw.at[npar], si)

            for c in range(W):
                p = c % 2
                wait_g(bufs[p], sgs[p])
                scat(didxw.at[wpar, c], bufs[p], sss[p])
                wait_s(bufs[p], sss[p])
                if c == 2:
                    @pl.when(w < NWIN - 1)
                    def _():
                        pltpu.make_async_copy(
                            srcr_hbm.at[pl.ds(0, W)], sidxw.at[0], si).wait()
                        pltpu.make_async_copy(
                            dstr_hbm.at[pl.ds(0, W)], didxw.at[0], si).wait()
                if c < W - 2:
                    gather(wpar, c + 2, bufs[p], sgs[p])
                else:
                    @pl.when(w < NWIN - 1)
                    def _():
                        gather(npar, c - 2, bufs[p], sgs[p])
            return carry

        lax.fori_loop(0, NWIN, wbody, 0)
        plsc.subcore_barrier()
        pltpu.sync_copy(acc.at[pl.ds(base, ROWS_T)],
                        out_hbm.at[cc, pl.ds(base, ROWS_T)])
        if half == 0:
            plsc.subcore_barrier()


def _prologue_body(nf_ref, degp_ref, w_in_ref, b_in_ref, w0_ref,
                   onorm_ref, inorm_ref, h4_ref, y4_ref):
    i = pl.program_id(0)
    dsrc = degp_ref[0, 0] + degp_ref[1, 0]
    ddst = degp_ref[0, 1] + degp_ref[1, 1]
    on = lax.rsqrt(jnp.maximum(dsrc, 1.0))
    inn = lax.rsqrt(jnp.maximum(ddst, 1.0))
    onorm_ref[...] = on
    inorm_ref[...] = inn
    rows = lax.broadcasted_iota(jnp.int32, (R, 1), 0) + i * R
    mask = (rows < N).astype(jnp.float32)
    h = mask * (jnp.dot(nf_ref[...], w_in_ref[...],
                        preferred_element_type=jnp.float32) + b_in_ref[...])
    y = jnp.dot(h * on, w0_ref[...], preferred_element_type=jnp.float32)
    for c in range(4):
        h4_ref[c] = h[:, c * 128:(c + 1) * 128]
        y4_ref[c] = y[:, c * 128:(c + 1) * 128]


def _tc_prologue(nf, degp, w_in, b_in, w0):
    return pl.pallas_call(
        _prologue_body,
        grid=(NB,),
        in_specs=[
            pl.BlockSpec((R, DIN), lambda i: (i, 0)),
            pl.BlockSpec((2, 2, R, 1), lambda i: (0, 0, i, 0)),
            pl.BlockSpec((DIN, H), lambda i: (0, 0)),
            pl.BlockSpec((1, H), lambda i: (0, 0)),
            pl.BlockSpec((H, H), lambda i: (0, 0)),
        ],
        out_specs=[
            pl.BlockSpec((R, 1), lambda i: (i, 0)),
            pl.BlockSpec((R, 1), lambda i: (i, 0)),
            pl.BlockSpec((4, R, 128), lambda i: (0, i, 0)),
            pl.BlockSpec((4, R, 128), lambda i: (0, i, 0)),
        ],
        out_shape=[
            jax.ShapeDtypeStruct((NP, 1), jnp.float32),
            jax.ShapeDtypeStruct((NP, 1), jnp.float32),
            jax.ShapeDtypeStruct((4, NP, 128), jnp.float32),
            jax.ShapeDtypeStruct((4, NP, 128), jnp.float32),
        ],
    )(nf, degp, w_in, b_in, w0)


def _stats_body(s4_ref, inorm_ref, stats_ref):
    i = pl.program_id(0)
    inn = inorm_ref[...]
    sus = []
    sqs = []
    for c in range(4):
        t = s4_ref[c] * inn
        sus.append(jnp.sum(t, axis=0, keepdims=True))
        sqs.append(jnp.sum(t * t, axis=0, keepdims=True))
    su = jnp.concatenate(sus, axis=1)
    sq = jnp.concatenate(sqs, axis=1)
    blk = jnp.concatenate([su, sq, jnp.zeros((6, H), jnp.float32)], axis=0)

    @pl.when(i == 0)
    def _():
        stats_ref[...] = blk

    @pl.when(i > 0)
    def _():
        stats_ref[...] += blk


def _tc_stats(s4, inorm):
    return pl.pallas_call(
        _stats_body,
        grid=(NB,),
        in_specs=[
            pl.BlockSpec((4, R, 128), lambda i: (0, i, 0)),
            pl.BlockSpec((R, 1), lambda i: (i, 0)),
        ],
        out_specs=pl.BlockSpec((8, H), lambda i: (0, 0)),
        out_shape=jax.ShapeDtypeStruct((8, H), jnp.float32),
    )(s4, inorm)


def _layer_body(is_res, has_next,
                s4_ref, stats_ref, inorm_ref, gamma_ref, beta_ref,
                h4_ref, onorm_ref, wn_ref, *out_refs):
    i = pl.program_id(0)
    su = stats_ref[0:1, :] * (1.0 / N)
    var = stats_ref[1:2, :] * (1.0 / N) - su * su
    scale = lax.rsqrt(var + 1e-5) * gamma_ref[...]
    beta = beta_ref[...]
    inn = inorm_ref[...]
    rows = lax.broadcasted_iota(jnp.int32, (R, 1), 0) + i * R
    mask = (rows < N).astype(jnp.float32)
    nxt = []
    for c in range(4):
        sl = slice(c * 128, (c + 1) * 128)
        t = s4_ref[c] * inn
        x = mask * jnp.maximum((t - su[:, sl]) * scale[:, sl] + beta[:, sl], 0.0)
        if is_res:
            hc = h4_ref[c] + x
            out_refs[0][c] = hc
            nxt.append(hc)
        else:
            nxt.append(x)
    if has_next:
        xin = jnp.concatenate(nxt, axis=1) * onorm_ref[...]
        y = jnp.dot(xin, wn_ref[...], preferred_element_type=jnp.float32)
        yref = out_refs[1] if is_res else out_refs[0]
        for c in range(4):
            yref[c] = y[:, c * 128:(c + 1) * 128]


def _tc_layer(s4, stats, inorm, gamma, beta, h4, onorm, wn, is_res, has_next):
    node_spec = pl.BlockSpec((4, R, 128), lambda i: (0, i, 0))
    node_shape = jax.ShapeDtypeStruct((4, NP, 128), jnp.float32)
    out_specs = []
    out_shape = []
    if is_res:
        out_specs.append(node_spec)
        out_shape.append(node_shape)
    if has_next:
        out_specs.append(node_spec)
        out_shape.append(node_shape)
    return pl.pallas_call(
        functools.partial(_layer_body, is_res, has_next),
        grid=(NB,),
        in_specs=[
            node_spec,
            pl.BlockSpec((8, H), lambda i: (0, 0)),
            pl.BlockSpec((R, 1), lambda i: (i, 0)),
            pl.BlockSpec((1, H), lambda i: (0, 0)),
            pl.BlockSpec((1, H), lambda i: (0, 0)),
            node_spec,
            pl.BlockSpec((R, 1), lambda i: (i, 0)),
            pl.BlockSpec((H, H), lambda i: (0, 0)),
        ],
        out_specs=out_specs,
        out_shape=out_shape,
    )(s4, stats, inorm, gamma, beta, h4, onorm, wn)


def _epilogue_body(h4_ref, wp_ref, bp_ref, wc_ref, bc_ref,
                   pose_ref, label_ref, hs_ref):
    i = pl.program_id(0)
    h = jnp.concatenate([h4_ref[c] for c in range(4)], axis=1)
    pose_ref[...] = jnp.dot(h, wp_ref[...],
                            preferred_element_type=jnp.float32) + bp_ref[...]
    hs = jnp.sum(h, axis=0, keepdims=True)
    blk = jnp.concatenate([hs, jnp.zeros((7, H), jnp.float32)], axis=0)

    @pl.when(i == 0)
    def _():
        hs_ref[...] = blk

    @pl.when(i > 0)
    def _():
        hs_ref[...] += blk

    @pl.when(i == NB - 1)
    def _():
        hmean = hs_ref[0:1, :] * (1.0 / N)
        lbl = jnp.dot(hmean, wc_ref[...],
                      preferred_element_type=jnp.float32) + bc_ref[...]
        label_ref[...] = jnp.concatenate(
            [lbl, jnp.zeros((7, 128), jnp.float32)], axis=0)


def _tc_epilogue(h4, wp, bp, wc, bc):
    return pl.pallas_call(
        _epilogue_body,
        grid=(NB,),
        in_specs=[
            pl.BlockSpec((4, R, 128), lambda i: (0, i, 0)),
            pl.BlockSpec((H, 128), lambda i: (0, 0)),
            pl.BlockSpec((1, 128), lambda i: (0, 0)),
            pl.BlockSpec((H, 128), lambda i: (0, 0)),
            pl.BlockSpec((1, 128), lambda i: (0, 0)),
        ],
        out_specs=[
            pl.BlockSpec((R, 128), lambda i: (i, 0)),
            pl.BlockSpec((8, 128), lambda i: (0, 0)),
        ],
        out_shape=[
            jax.ShapeDtypeStruct((NP, 128), jnp.float32),
            jax.ShapeDtypeStruct((8, 128), jnp.float32),
        ],
        scratch_shapes=[pltpu.VMEM((8, H), jnp.float32)],
    )(h4, wp, bp, wc, bc)


def kernel(node_features, edge_index, W_in, b_in, W_conv, b_conv, gamma, beta,
           W_pose, b_pose, W_cls, b_cls):
    f32 = jnp.float32
    pad_e = EP - E
    srcp = jnp.concatenate(
        [edge_index[0], jnp.full((pad_e,), PAD_NODE, jnp.int32)]).reshape(EP // K, K)
    dstp = jnp.concatenate(
        [edge_index[1], jnp.full((pad_e,), PAD_NODE, jnp.int32)]).reshape(EP // K, K)
    srcd = srcp.reshape(EP // KD, KD)
    dstd = dstp.reshape(EP // KD, KD)
    nf = jnp.pad(node_features, ((0, NP - N), (0, 0)))
    ones_kd = jnp.ones((KD,), f32)
    zrow = jnp.zeros((ROWS_T,), f32)
    zblk = jnp.zeros((ROWS_T, 128), f32)

    degp = _sc_degrees(srcd, dstd, ones_kd, zrow)
    degp4 = degp.reshape(2, 2, NP, 1)

    onorm, inorm, h4, y4 = _tc_prologue(
        nf, degp4, W_in, b_in.reshape(1, H), W_conv[0])

    for i in range(NL):
        s4 = _sc_aggregate(y4, srcp, dstp, zblk)
        stats = _tc_stats(s4, inorm)
        is_res = (i % 2 == 1)
        has_next = (i < NL - 1)
        wn = W_conv[i + 1] if has_next else W_conv[i]
        outs = _tc_layer(s4, stats, inorm,
                         gamma[i].reshape(1, H), beta[i].reshape(1, H),
                         h4, onorm, wn, is_res, has_next)
        if is_res and has_next:
            h4, y4 = outs
        elif is_res:
            h4 = outs[0] if isinstance(outs, (list, tuple)) else outs
        else:
            y4 = outs[0] if isinstance(outs, (list, tuple)) else outs

    wp = jnp.pad(W_pose, ((0, 0), (0, 128 - W_pose.shape[1])))
    bp = jnp.pad(b_pose, (0, 128 - b_pose.shape[0])).reshape(1, 128)
    wc = jnp.pad(W_cls, ((0, 0), (0, 128 - W_cls.shape[1])))
    bc = jnp.pad(b_cls, (0, 128 - b_cls.shape[0])).reshape(1, 128)
    pose_pad, label_pad = _tc_epilogue(h4, wp, bp, wc, bc)
    pose = pose_pad[:N, :W_pose.shape[1]]
    label = label_pad[0:1, :W_cls.shape[1]]
    return (pose, label)


# speedup vs baseline: 1.0001x; 1.0001x over previous
import functools

import jax
import jax.numpy as jnp
from jax import lax
from jax.experimental import pallas as pl
from jax.experimental.pallas import tpu as pltpu
from jax.experimental.pallas import tpu_sc as plsc

N = 10000
NP = 10240
E = 160000
EP = 163840
DIN = 256
H = 512
NL = 12
PAD_NODE = 10200

K = 128
EPW = EP // 16
NCH = EPW // K
ROWS_T = NP // 16

KD = 128
EPW2 = EP // 32
NCHD = EPW2 // KD
W = 4
NWIN = NCH // W

R = 1024
NB = NP // R

_mesh = plsc.VectorSubcoreMesh(core_axis_name="c", subcore_axis_name="s")


@functools.partial(
    pl.kernel,
    out_type=jax.ShapeDtypeStruct((2, 2, NP), jnp.float32),
    mesh=_mesh,
    scratch_types=[
        pltpu.VMEM_SHARED((NP,), jnp.float32),
        pltpu.VMEM_SHARED((NP,), jnp.float32),
        pltpu.VMEM((NCHD, KD), jnp.int32),
        pltpu.VMEM((NCHD, KD), jnp.int32),
        pltpu.VMEM((KD,), jnp.float32),
        pltpu.VMEM((ROWS_T,), jnp.float32),
    ],
)
def _sc_degrees(src_hbm, dst_hbm, ones_hbm, zrow_hbm, out_hbm,
                acc_s, acc_d, sidx, didx, ones_v, zrow_v):
    cid = lax.axis_index("c")
    sid = lax.axis_index("s")
    wid = sid * 2 + cid
    pltpu.sync_copy(ones_hbm, ones_v)
    pltpu.sync_copy(zrow_hbm, zrow_v)
    pltpu.sync_copy(src_hbm.at[pl.ds(wid * NCHD, NCHD)], sidx)
    pltpu.sync_copy(dst_hbm.at[pl.ds(wid * NCHD, NCHD)], didx)
    pltpu.sync_copy(zrow_v, acc_s.at[pl.ds(sid * ROWS_T, ROWS_T)])
    pltpu.sync_copy(zrow_v, acc_d.at[pl.ds(sid * ROWS_T, ROWS_T)])
    plsc.subcore_barrier()

    def body(j, carry):
        pltpu.sync_copy(ones_v, acc_s.at[sidx.at[j]], add=True)
        pltpu.sync_copy(ones_v, acc_d.at[didx.at[j]], add=True)
        return carry

    lax.fori_loop(0, NCHD, body, 0)
    plsc.subcore_barrier()
    pltpu.sync_copy(acc_s.at[pl.ds(sid * ROWS_T, ROWS_T)],
                    out_hbm.at[cid, 0, pl.ds(sid * ROWS_T, ROWS_T)])
    pltpu.sync_copy(acc_d.at[pl.ds(sid * ROWS_T, ROWS_T)],
                    out_hbm.at[cid, 1, pl.ds(sid * ROWS_T, ROWS_T)])


@functools.partial(
    pl.kernel,
    out_type=jax.ShapeDtypeStruct((4, NP, 128), jnp.float32),
    mesh=_mesh,
    scratch_types=[
        pltpu.VMEM_SHARED((NP, 128), jnp.float32),
        pltpu.VMEM((2, W, K), jnp.int32),
        pltpu.VMEM((2, W, K), jnp.int32),
        pltpu.VMEM((K, 128), jnp.float32),
        pltpu.VMEM((K, 128), jnp.float32),
        pltpu.SemaphoreType.DMA,
        pltpu.SemaphoreType.DMA,
        pltpu.SemaphoreType.DMA,
        pltpu.SemaphoreType.DMA,
        pltpu.SemaphoreType.DMA,
    ],
)
def _sc_aggregate(y_hbm, srcr_hbm, dstr_hbm, z_hbm, out_hbm,
                  acc, sidxw, didxw, buf0, buf1, sg0, sg1, ss0, ss1, si):
    cid = lax.axis_index("c")
    sid = lax.axis_index("s")
    base = sid * ROWS_T
    bufs = (buf0, buf1)
    sgs = (sg0, sg1)
    sss = (ss0, ss1)
    for half in range(2):
        cc = cid * 2 + half
        pltpu.sync_copy(z_hbm, acc.at[pl.ds(base, ROWS_T)])
        pltpu.sync_copy(srcr_hbm.at[pl.ds(sid * NCH, W)], sidxw.at[0])
        pltpu.sync_copy(dstr_hbm.at[pl.ds(sid * NCH, W)], didxw.at[0])
        plsc.subcore_barrier()

        def gather(wp, ci, buf, sem):
            return pltpu.async_copy(y_hbm.at[cc].at[sidxw.at[wp, ci]],
                                    buf, sem)

        def scat(idx, buf, sem):
            return pltpu.async_copy(buf, acc.at[idx], sem, add=True)

        def wait_g(buf, sem):
            pltpu.make_async_copy(y_hbm.at[cc].at[sidxw.at[0, 0]], buf,
                                  sem).wait()

        def wait_s(buf, sem):
            pltpu.make_async_copy(buf, acc.at[didxw.at[0, 0]], sem).wait()

        gather(0, 0, buf0, sg0)
        gather(0, 1, buf1, sg1)

        def wbody(w, carry):
            wpar = w % 2
            npar = 1 - wpar

            @pl.when(w < NWIN - 1)
            def _():
                pltpu.async_copy(
                    srcr_hbm.at[pl.ds(sid * NCH + (w + 1) * W, W)],
                    sidxw.at[npar], si)
                pltpu.async_copy(
                    dstr_hbm.at[pl.ds(sid * NCH + (w + 1) * W, W)],
                    didxw.at[npar], si)

            for c in range(W):
                p = c % 2
                wait_g(bufs[p], sgs[p])
                scat(didxw.at[wpar, c], bufs[p], sss[p])
                wait_s(bufs[p], sss[p])
                if c == 2:
                    @pl.when(w < NWIN - 1)
                    def _():
                        pltpu.make_async_copy(
                            srcr_hbm.at[pl.ds(0, W)], sidxw.at[0], si).wait()
                        pltpu.make_async_copy(
                            dstr_hbm.at[pl.ds(0, W)], didxw.at[0], si).wait()
                if c < W - 2:
                    gather(wpar, c + 2, bufs[p], sgs[p])
                else:
                    @pl.when(w < NWIN - 1)
                    def _():
                        gather(npar, c - 2, bufs[p], sgs[p])
            return carry

        lax.fori_loop(0, NWIN, wbody, 0)
        plsc.subcore_barrier()
        pltpu.sync_copy(acc.at[pl.ds(base, ROWS_T)],
                        out_hbm.at[cc, pl.ds(base, ROWS_T)])
        if half == 0:
            plsc.subcore_barrier()


def _prologue_body(nf_ref, degp_ref, w_in_ref, b_in_ref, w0_ref,
                   onorm_ref, inorm_ref, h4_ref, y4_ref):
    i = pl.program_id(0)
    dsrc = degp_ref[0, 0] + degp_ref[1, 0]
    ddst = degp_ref[0, 1] + degp_ref[1, 1]
    on = lax.rsqrt(jnp.maximum(dsrc, 1.0))
    inn = lax.rsqrt(jnp.maximum(ddst, 1.0))
    onorm_ref[...] = on
    inorm_ref[...] = inn
    rows = lax.broadcasted_iota(jnp.int32, (R, 1), 0) + i * R
    mask = (rows < N).astype(jnp.float32)
    h = mask * (jnp.dot(nf_ref[...], w_in_ref[...],
                        preferred_element_type=jnp.float32) + b_in_ref[...])
    y = jnp.dot(h * on, w0_ref[...], preferred_element_type=jnp.float32)
    for c in range(4):
        h4_ref[c] = h[:, c * 128:(c + 1) * 128]
        y4_ref[c] = y[:, c * 128:(c + 1) * 128]


def _tc_prologue(nf, degp, w_in, b_in, w0):
    return pl.pallas_call(
        _prologue_body,
        grid=(NB,),
        in_specs=[
            pl.BlockSpec((R, DIN), lambda i: (i, 0)),
            pl.BlockSpec((2, 2, R, 1), lambda i: (0, 0, i, 0)),
            pl.BlockSpec((DIN, H), lambda i: (0, 0)),
            pl.BlockSpec((1, H), lambda i: (0, 0)),
            pl.BlockSpec((H, H), lambda i: (0, 0)),
        ],
        out_specs=[
            pl.BlockSpec((R, 1), lambda i: (i, 0)),
            pl.BlockSpec((R, 1), lambda i: (i, 0)),
            pl.BlockSpec((4, R, 128), lambda i: (0, i, 0)),
            pl.BlockSpec((4, R, 128), lambda i: (0, i, 0)),
        ],
        out_shape=[
            jax.ShapeDtypeStruct((NP, 1), jnp.float32),
            jax.ShapeDtypeStruct((NP, 1), jnp.float32),
            jax.ShapeDtypeStruct((4, NP, 128), jnp.float32),
            jax.ShapeDtypeStruct((4, NP, 128), jnp.float32),
        ],
    )(nf, degp, w_in, b_in, w0)


def _stats_body(s4_ref, inorm_ref, stats_ref):
    i = pl.program_id(0)
    inn = inorm_ref[...]
    sus = []
    sqs = []
    for c in range(4):
        t = s4_ref[c] * inn
        sus.append(jnp.sum(t, axis=0, keepdims=True))
        sqs.append(jnp.sum(t * t, axis=0, keepdims=True))
    su = jnp.concatenate(sus, axis=1)
    sq = jnp.concatenate(sqs, axis=1)
    blk = jnp.concatenate([su, sq, jnp.zeros((6, H), jnp.float32)], axis=0)

    @pl.when(i == 0)
    def _():
        stats_ref[...] = blk

    @pl.when(i > 0)
    def _():
        stats_ref[...] += blk


def _tc_stats(s4, inorm):
    return pl.pallas_call(
        _stats_body,
        grid=(NB,),
        in_specs=[
            pl.BlockSpec((4, R, 128), lambda i: (0, i, 0)),
            pl.BlockSpec((R, 1), lambda i: (i, 0)),
        ],
        out_specs=pl.BlockSpec((8, H), lambda i: (0, 0)),
        out_shape=jax.ShapeDtypeStruct((8, H), jnp.float32),
    )(s4, inorm)


def _layer_body(is_res, has_next,
                s4_ref, stats_ref, inorm_ref, gamma_ref, beta_ref,
                h4_ref, onorm_ref, wn_ref, *out_refs):
    i = pl.program_id(0)
    su = stats_ref[0:1, :] * (1.0 / N)
    var = stats_ref[1:2, :] * (1.0 / N) - su * su
    scale = lax.rsqrt(var + 1e-5) * gamma_ref[...]
    beta = beta_ref[...]
    inn = inorm_ref[...]
    rows = lax.broadcasted_iota(jnp.int32, (R, 1), 0) + i * R
    mask = (rows < N).astype(jnp.float32)
    nxt = []
    for c in range(4):
        sl = slice(c * 128, (c + 1) * 128)
        t = s4_ref[c] * inn
        x = mask * jnp.maximum((t - su[:, sl]) * scale[:, sl] + beta[:, sl], 0.0)
        if is_res:
            hc = h4_ref[c] + x
            out_refs[0][c] = hc
            nxt.append(hc)
        else:
            nxt.append(x)
    if has_next:
        xin = jnp.concatenate(nxt, axis=1) * onorm_ref[...]
        y = jnp.dot(xin, wn_ref[...], preferred_element_type=jnp.float32)
        yref = out_refs[1] if is_res else out_refs[0]
        for c in range(4):
            yref[c] = y[:, c * 128:(c + 1) * 128]


def _tc_layer(s4, stats, inorm, gamma, beta, h4, onorm, wn, is_res, has_next):
    node_spec = pl.BlockSpec((4, R, 128), lambda i: (0, i, 0))
    node_shape = jax.ShapeDtypeStruct((4, NP, 128), jnp.float32)
    out_specs = []
    out_shape = []
    if is_res:
        out_specs.append(node_spec)
        out_shape.append(node_shape)
    if has_next:
        out_specs.append(node_spec)
        out_shape.append(node_shape)
    return pl.pallas_call(
        functools.partial(_layer_body, is_res, has_next),
        grid=(NB,),
        in_specs=[
            node_spec,
            pl.BlockSpec((8, H), lambda i: (0, 0)),
            pl.BlockSpec((R, 1), lambda i: (i, 0)),
            pl.BlockSpec((1, H), lambda i: (0, 0)),
            pl.BlockSpec((1, H), lambda i: (0, 0)),
            node_spec,
            pl.BlockSpec((R, 1), lambda i: (i, 0)),
            pl.BlockSpec((H, H), lambda i: (0, 0)),
        ],
        out_specs=out_specs,
        out_shape=out_shape,
    )(s4, stats, inorm, gamma, beta, h4, onorm, wn)


def _epilogue_body(h4_ref, wp_ref, bp_ref, wc_ref, bc_ref,
                   pose_ref, label_ref, hs_ref):
    i = pl.program_id(0)
    h = jnp.concatenate([h4_ref[c] for c in range(4)], axis=1)
    pose_ref[...] = jnp.dot(h, wp_ref[...],
                            preferred_element_type=jnp.float32) + bp_ref[...]
    hs = jnp.sum(h, axis=0, keepdims=True)
    blk = jnp.concatenate([hs, jnp.zeros((7, H), jnp.float32)], axis=0)

    @pl.when(i == 0)
    def _():
        hs_ref[...] = blk

    @pl.when(i > 0)
    def _():
        hs_ref[...] += blk

    @pl.when(i == NB - 1)
    def _():
        hmean = hs_ref[0:1, :] * (1.0 / N)
        lbl = jnp.dot(hmean, wc_ref[...],
                      preferred_element_type=jnp.float32) + bc_ref[...]
        label_ref[...] = jnp.concatenate(
            [lbl, jnp.zeros((7, 128), jnp.float32)], axis=0)


def _tc_epilogue(h4, wp, bp, wc, bc):
    return pl.pallas_call(
        _epilogue_body,
        grid=(NB,),
        in_specs=[
            pl.BlockSpec((4, R, 128), lambda i: (0, i, 0)),
            pl.BlockSpec((H, 128), lambda i: (0, 0)),
            pl.BlockSpec((1, 128), lambda i: (0, 0)),
            pl.BlockSpec((H, 128), lambda i: (0, 0)),
            pl.BlockSpec((1, 128), lambda i: (0, 0)),
        ],
        out_specs=[
            pl.BlockSpec((R, 128), lambda i: (i, 0)),
            pl.BlockSpec((8, 128), lambda i: (0, 0)),
        ],
        out_shape=[
            jax.ShapeDtypeStruct((NP, 128), jnp.float32),
            jax.ShapeDtypeStruct((8, 128), jnp.float32),
        ],
        scratch_shapes=[pltpu.VMEM((8, H), jnp.float32)],
    )(h4, wp, bp, wc, bc)


def kernel(node_features, edge_index, W_in, b_in, W_conv, b_conv, gamma, beta,
           W_pose, b_pose, W_cls, b_cls):
    f32 = jnp.float32
    pad_e = EP - E
    srcp = jnp.concatenate(
        [edge_index[0], jnp.full((pad_e,), PAD_NODE, jnp.int32)]).reshape(EP // K, K)
    dstp = jnp.concatenate(
        [edge_index[1], jnp.full((pad_e,), PAD_NODE, jnp.int32)]).reshape(EP // K, K)
    srcd = srcp.reshape(EP // KD, KD)
    dstd = dstp.reshape(EP // KD, KD)
    nf = jnp.pad(node_features, ((0, NP - N), (0, 0)))
    ones_kd = jnp.ones((KD,), f32)
    zrow = jnp.zeros((ROWS_T,), f32)
    zblk = jnp.zeros((ROWS_T, 128), f32)

    degp = _sc_degrees(srcd, dstd, ones_kd, zrow)
    degp4 = degp.reshape(2, 2, NP, 1)

    onorm, inorm, h4, y4 = _tc_prologue(
        nf, degp4, W_in, b_in.reshape(1, H), W_conv[0])

    for i in range(NL):
        s4 = _sc_aggregate(y4, srcp, dstp, zblk)
        stats = _tc_stats(s4, inorm)
        is_res = (i % 2 == 1)
        has_next = (i < NL - 1)
        wn = W_conv[i + 1] if has_next else W_conv[i]
        outs = _tc_layer(s4, stats, inorm,
                         gamma[i].reshape(1, H), beta[i].reshape(1, H),
                         h4, onorm, wn, is_res, has_next)
        if is_res and has_next:
            h4, y4 = outs
        elif is_res:
            h4 = outs[0] if isinstance(outs, (list, tuple)) else outs
        else:
            y4 = outs[0] if isinstance(outs, (list, tuple)) else outs

    wp = jnp.pad(W_pose, ((0, 0), (0, 128 - W_pose.shape[1])))
    bp = jnp.pad(b_pose, (0, 128 - b_pose.shape[0])).reshape(1, 128)
    wc = jnp.pad(W_cls, ((0, 0), (0, 128 - W_cls.shape[1])))
    bc = jnp.pad(b_cls, (0, 128 - b_cls.shape[0])).reshape(1, 128)
    pose_pad, label_pad = _tc_epilogue(h4, wp, bp, wc, bc)
    pose = pose_pad[:N, :W_pose.shape[1]]
    label = label_pad[0:1, :W_cls.shape[1]]
    return (pose, label)


# speedup vs baseline: 1.0962x; 1.0961x over previous
import functools

import jax
import jax.numpy as jnp
from jax import lax
from jax.experimental import pallas as pl
from jax.experimental.pallas import tpu as pltpu
from jax.experimental.pallas import tpu_sc as plsc

N = 10000
NP = 10240
E = 160000
EP = 163840
DIN = 256
H = 512
NL = 12
PAD_NODE = 10200

K = 128
EPW = EP // 16
NCH = EPW // K
ROWS_T = NP // 16

KD = 128
EPW2 = EP // 32
NCHD = EPW2 // KD
W = 4
NWIN = NCH // W

R = 1024
NB = NP // R

_mesh = plsc.VectorSubcoreMesh(core_axis_name="c", subcore_axis_name="s")


@functools.partial(
    pl.kernel,
    out_type=jax.ShapeDtypeStruct((2, 2, NP), jnp.float32),
    mesh=_mesh,
    scratch_types=[
        pltpu.VMEM_SHARED((NP,), jnp.float32),
        pltpu.VMEM_SHARED((NP,), jnp.float32),
        pltpu.VMEM((NCHD, KD), jnp.int32),
        pltpu.VMEM((NCHD, KD), jnp.int32),
        pltpu.VMEM((KD,), jnp.float32),
        pltpu.VMEM((ROWS_T,), jnp.float32),
    ],
)
def _sc_degrees(src_hbm, dst_hbm, ones_hbm, zrow_hbm, out_hbm,
                acc_s, acc_d, sidx, didx, ones_v, zrow_v):
    cid = lax.axis_index("c")
    sid = lax.axis_index("s")
    wid = sid * 2 + cid
    pltpu.sync_copy(ones_hbm, ones_v)
    pltpu.sync_copy(zrow_hbm, zrow_v)
    pltpu.sync_copy(src_hbm.at[pl.ds(wid * NCHD, NCHD)], sidx)
    pltpu.sync_copy(dst_hbm.at[pl.ds(wid * NCHD, NCHD)], didx)
    pltpu.sync_copy(zrow_v, acc_s.at[pl.ds(sid * ROWS_T, ROWS_T)])
    pltpu.sync_copy(zrow_v, acc_d.at[pl.ds(sid * ROWS_T, ROWS_T)])
    plsc.subcore_barrier()

    def body(j, carry):
        pltpu.sync_copy(ones_v, acc_s.at[sidx.at[j]], add=True)
        pltpu.sync_copy(ones_v, acc_d.at[didx.at[j]], add=True)
        return carry

    lax.fori_loop(0, NCHD, body, 0)
    plsc.subcore_barrier()
    pltpu.sync_copy(acc_s.at[pl.ds(sid * ROWS_T, ROWS_T)],
                    out_hbm.at[cid, 0, pl.ds(sid * ROWS_T, ROWS_T)])
    pltpu.sync_copy(acc_d.at[pl.ds(sid * ROWS_T, ROWS_T)],
                    out_hbm.at[cid, 1, pl.ds(sid * ROWS_T, ROWS_T)])


@functools.partial(
    pl.kernel,
    out_type=jax.ShapeDtypeStruct((4, NP, 128), jnp.float32),
    mesh=_mesh,
    scratch_types=[
        pltpu.VMEM_SHARED((NP, 128), jnp.float32),
        pltpu.VMEM((2, W, K), jnp.int32),
        pltpu.VMEM((2, W, K), jnp.int32),
        pltpu.VMEM((K, 128), jnp.float32),
        pltpu.VMEM((K, 128), jnp.float32),
        pltpu.SemaphoreType.DMA,
        pltpu.SemaphoreType.DMA,
        pltpu.SemaphoreType.DMA,
        pltpu.SemaphoreType.DMA,
        pltpu.SemaphoreType.DMA,
    ],
)
def _sc_aggregate(y_hbm, srcr_hbm, dstr_hbm, z_hbm, out_hbm,
                  acc, sidxw, didxw, buf0, buf1, sg0, sg1, ss0, ss1, si):
    cid = lax.axis_index("c")
    sid = lax.axis_index("s")
    base = sid * ROWS_T
    bufs = (buf0, buf1)
    sgs = (sg0, sg1)
    sss = (ss0, ss1)
    for half in range(2):
        cc = cid * 2 + half
        pltpu.sync_copy(z_hbm, acc.at[pl.ds(base, ROWS_T)])
        pltpu.sync_copy(srcr_hbm.at[pl.ds(sid * NCH, W)], sidxw.at[0])
        pltpu.sync_copy(dstr_hbm.at[pl.ds(sid * NCH, W)], didxw.at[0])
        plsc.subcore_barrier()

        def gather(wp, ci, buf, sem):
            return pltpu.async_copy(y_hbm.at[cc].at[sidxw.at[wp, ci]],
                                    buf, sem)

        def scat(idx, buf, sem):
            return pltpu.async_copy(buf, acc.at[idx], sem, add=True)

        def wait_g(buf, sem):
            pltpu.make_async_copy(y_hbm.at[cc].at[sidxw.at[0, 0]], buf,
                                  sem).wait()

        def wait_s(buf, sem):
            pltpu.make_async_copy(buf, acc.at[didxw.at[0, 0]], sem).wait()

        gather(0, 0, buf0, sg0)
        gather(0, 1, buf1, sg1)

        def wbody(w, carry):
            wpar = w % 2
            npar = 1 - wpar

            @pl.when(w < NWIN - 1)
            def _():
                pltpu.async_copy(
                    srcr_hbm.at[pl.ds(sid * NCH + (w + 1) * W, W)],
                    sidxw.at[npar], si)
                pltpu.async_copy(
                    dstr_hbm.at[pl.ds(sid * NCH + (w + 1) * W, W)],
                    didxw.at[npar], si)

            for c in range(W):
                p = c % 2
                wait_g(bufs[p], sgs[p])
                scat(didxw.at[wpar, c], bufs[p], sss[p])
                wait_s(bufs[p], sss[p])
                if c == 2:
                    @pl.when(w < NWIN - 1)
                    def _():
                        pltpu.make_async_copy(
                            srcr_hbm.at[pl.ds(0, W)], sidxw.at[0], si).wait()
                        pltpu.make_async_copy(
                            dstr_hbm.at[pl.ds(0, W)], didxw.at[0], si).wait()
                if c < W - 2:
                    gather(wpar, c + 2, bufs[p], sgs[p])
                else:
                    @pl.when(w < NWIN - 1)
                    def _():
                        gather(npar, c - 2, bufs[p], sgs[p])
            return carry

        lax.fori_loop(0, NWIN, wbody, 0)
        plsc.subcore_barrier()
        pltpu.sync_copy(acc.at[pl.ds(base, ROWS_T)],
                        out_hbm.at[cc, pl.ds(base, ROWS_T)])
        if half == 0:
            plsc.subcore_barrier()


def _prologue_body(nf_ref, degp_ref, w_in_ref, b_in_ref,
                   onorm_ref, inorm_ref, h4_ref, f4_ref):
    i = pl.program_id(0)
    dsrc = degp_ref[0, 0] + degp_ref[1, 0]
    ddst = degp_ref[0, 1] + degp_ref[1, 1]
    on = lax.rsqrt(jnp.maximum(dsrc, 1.0))
    inn = lax.rsqrt(jnp.maximum(ddst, 1.0))
    onorm_ref[...] = on
    inorm_ref[...] = inn
    rows = lax.broadcasted_iota(jnp.int32, (R, 1), 0) + i * R
    mask = (rows < N).astype(jnp.float32)
    h = mask * (jnp.dot(nf_ref[...], w_in_ref[...],
                        preferred_element_type=jnp.float32) + b_in_ref[...])
    f = h * on
    for c in range(4):
        h4_ref[c] = h[:, c * 128:(c + 1) * 128]
        f4_ref[c] = f[:, c * 128:(c + 1) * 128]


def _tc_prologue(nf, degp, w_in, b_in):
    return pl.pallas_call(
        _prologue_body,
        grid=(NB,),
        in_specs=[
            pl.BlockSpec((R, DIN), lambda i: (i, 0)),
            pl.BlockSpec((2, 2, R, 1), lambda i: (0, 0, i, 0)),
            pl.BlockSpec((DIN, H), lambda i: (0, 0)),
            pl.BlockSpec((1, H), lambda i: (0, 0)),
        ],
        out_specs=[
            pl.BlockSpec((R, 1), lambda i: (i, 0)),
            pl.BlockSpec((R, 1), lambda i: (i, 0)),
            pl.BlockSpec((4, R, 128), lambda i: (0, i, 0)),
            pl.BlockSpec((4, R, 128), lambda i: (0, i, 0)),
        ],
        out_shape=[
            jax.ShapeDtypeStruct((NP, 1), jnp.float32),
            jax.ShapeDtypeStruct((NP, 1), jnp.float32),
            jax.ShapeDtypeStruct((4, NP, 128), jnp.float32),
            jax.ShapeDtypeStruct((4, NP, 128), jnp.float32),
        ],
    )(nf, degp, w_in, b_in)


def _mm_body(s4_ref, inorm_ref, b_ref, w_ref, z4_ref, stats_ref):
    i = pl.program_id(0)
    inn = inorm_ref[...]
    a = jnp.concatenate([s4_ref[c] * inn for c in range(4)], axis=1)
    z = jnp.dot(a, w_ref[...], preferred_element_type=jnp.float32) + b_ref[...]
    rows = lax.broadcasted_iota(jnp.int32, (R, 1), 0) + i * R
    mask = (rows < N).astype(jnp.float32)
    zm = mask * z
    for c in range(4):
        z4_ref[c] = z[:, c * 128:(c + 1) * 128]
    blk = jnp.concatenate(
        [jnp.sum(zm, axis=0, keepdims=True),
         jnp.zeros((7, H), jnp.float32)], axis=0)

    @pl.when(i == 0)
    def _():
        stats_ref[...] = blk

    @pl.when(i > 0)
    def _():
        stats_ref[...] += blk


def _tc_mm(s4, inorm, b, w):
    return pl.pallas_call(
        _mm_body,
        grid=(NB,),
        in_specs=[
            pl.BlockSpec((4, R, 128), lambda i: (0, i, 0)),
            pl.BlockSpec((R, 1), lambda i: (i, 0)),
            pl.BlockSpec((1, H), lambda i: (0, 0)),
            pl.BlockSpec((H, H), lambda i: (0, 0)),
        ],
        out_specs=[
            pl.BlockSpec((4, R, 128), lambda i: (0, i, 0)),
            pl.BlockSpec((8, H), lambda i: (0, 0)),
        ],
        out_shape=[
            jax.ShapeDtypeStruct((4, NP, 128), jnp.float32),
            jax.ShapeDtypeStruct((8, H), jnp.float32),
        ],
    )(s4, inorm, b, w)


def _stats2_body(z4_ref, stats1_ref, var_ref):
    i = pl.program_id(0)
    m = stats1_ref[0:1, :] * (1.0 / N)
    rows = lax.broadcasted_iota(jnp.int32, (R, 1), 0) + i * R
    mask = (rows < N).astype(jnp.float32)
    sqs = []
    for c in range(4):
        sl = slice(c * 128, (c + 1) * 128)
        d = mask * (z4_ref[c] - m[:, sl])
        sqs.append(jnp.sum(d * d, axis=0, keepdims=True))
    sq = jnp.concatenate(sqs, axis=1)
    blk = jnp.concatenate([sq, jnp.zeros((7, H), jnp.float32)], axis=0)

    @pl.when(i == 0)
    def _():
        var_ref[...] = blk

    @pl.when(i > 0)
    def _():
        var_ref[...] += blk


def _tc_stats2(z4, stats1):
    return pl.pallas_call(
        _stats2_body,
        grid=(NB,),
        in_specs=[
            pl.BlockSpec((4, R, 128), lambda i: (0, i, 0)),
            pl.BlockSpec((8, H), lambda i: (0, 0)),
        ],
        out_specs=pl.BlockSpec((8, H), lambda i: (0, 0)),
        out_shape=jax.ShapeDtypeStruct((8, H), jnp.float32),
    )(z4, stats1)


def _layer_body(is_res, has_next,
                z4_ref, stats_ref, var_ref, gamma_ref, beta_ref,
                h4_ref, onorm_ref, *out_refs):
    i = pl.program_id(0)
    su = stats_ref[0:1, :] * (1.0 / N)
    var = var_ref[0:1, :] * (1.0 / N)
    scale = gamma_ref[...] / jnp.sqrt(var + 1e-5)
    beta = beta_ref[...]
    on = onorm_ref[...]
    rows = lax.broadcasted_iota(jnp.int32, (R, 1), 0) + i * R
    mask = (rows < N).astype(jnp.float32)
    for c in range(4):
        sl = slice(c * 128, (c + 1) * 128)
        x = mask * jnp.maximum(
            (z4_ref[c] - su[:, sl]) * scale[:, sl] + beta[:, sl], 0.0)
        if is_res:
            hc = h4_ref[c] + x
            out_refs[0][c] = hc
            nxt_c = hc
        else:
            nxt_c = x
        if has_next:
            fref = out_refs[1] if is_res else out_refs[0]
            fref[c] = nxt_c * on


def _tc_layer(z4, stats, var2, gamma, beta, h4, onorm, is_res, has_next):
    node_spec = pl.BlockSpec((4, R, 128), lambda i: (0, i, 0))
    node_shape = jax.ShapeDtypeStruct((4, NP, 128), jnp.float32)
    out_specs = []
    out_shape = []
    if is_res:
        out_specs.append(node_spec)
        out_shape.append(node_shape)
    if has_next:
        out_specs.append(node_spec)
        out_shape.append(node_shape)
    return pl.pallas_call(
        functools.partial(_layer_body, is_res, has_next),
        grid=(NB,),
        in_specs=[
            node_spec,
            pl.BlockSpec((8, H), lambda i: (0, 0)),
            pl.BlockSpec((8, H), lambda i: (0, 0)),
            pl.BlockSpec((1, H), lambda i: (0, 0)),
            pl.BlockSpec((1, H), lambda i: (0, 0)),
            node_spec,
            pl.BlockSpec((R, 1), lambda i: (i, 0)),
        ],
        out_specs=out_specs,
        out_shape=out_shape,
    )(z4, stats, var2, gamma, beta, h4, onorm)


def _epilogue_body(h4_ref, wp_ref, bp_ref, wc_ref, bc_ref,
                   pose_ref, label_ref, hs_ref):
    i = pl.program_id(0)
    h = jnp.concatenate([h4_ref[c] for c in range(4)], axis=1)
    pose_ref[...] = jnp.dot(h, wp_ref[...],
                            preferred_element_type=jnp.float32) + bp_ref[...]
    hs = jnp.sum(h, axis=0, keepdims=True)
    blk = jnp.concatenate([hs, jnp.zeros((7, H), jnp.float32)], axis=0)

    @pl.when(i == 0)
    def _():
        hs_ref[...] = blk

    @pl.when(i > 0)
    def _():
        hs_ref[...] += blk

    @pl.when(i == NB - 1)
    def _():
        hmean = hs_ref[0:1, :] * (1.0 / N)
        lbl = jnp.dot(hmean, wc_ref[...],
                      preferred_element_type=jnp.float32) + bc_ref[...]
        label_ref[...] = jnp.concatenate(
            [lbl, jnp.zeros((7, 128), jnp.float32)], axis=0)


def _tc_epilogue(h4, wp, bp, wc, bc):
    return pl.pallas_call(
        _epilogue_body,
        grid=(NB,),
        in_specs=[
            pl.BlockSpec((4, R, 128), lambda i: (0, i, 0)),
            pl.BlockSpec((H, 128), lambda i: (0, 0)),
            pl.BlockSpec((1, 128), lambda i: (0, 0)),
            pl.BlockSpec((H, 128), lambda i: (0, 0)),
            pl.BlockSpec((1, 128), lambda i: (0, 0)),
        ],
        out_specs=[
            pl.BlockSpec((R, 128), lambda i: (i, 0)),
            pl.BlockSpec((8, 128), lambda i: (0, 0)),
        ],
        out_shape=[
            jax.ShapeDtypeStruct((NP, 128), jnp.float32),
            jax.ShapeDtypeStruct((8, 128), jnp.float32),
        ],
        scratch_shapes=[pltpu.VMEM((8, H), jnp.float32)],
    )(h4, wp, bp, wc, bc)


def kernel(node_features, edge_index, W_in, b_in, W_conv, b_conv, gamma, beta,
           W_pose, b_pose, W_cls, b_cls):
    f32 = jnp.float32
    pad_e = EP - E
    srcp = jnp.concatenate(
        [edge_index[0], jnp.full((pad_e,), PAD_NODE, jnp.int32)]).reshape(EP // K, K)
    dstp = jnp.concatenate(
        [edge_index[1], jnp.full((pad_e,), PAD_NODE, jnp.int32)]).reshape(EP // K, K)
    srcd = srcp.reshape(EP // KD, KD)
    dstd = dstp.reshape(EP // KD, KD)
    nf = jnp.pad(node_features, ((0, NP - N), (0, 0)))
    ones_kd = jnp.ones((KD,), f32)
    zrow = jnp.zeros((ROWS_T,), f32)
    zblk = jnp.zeros((ROWS_T, 128), f32)

    degp = _sc_degrees(srcd, dstd, ones_kd, zrow)
    degp4 = degp.reshape(2, 2, NP, 1)

    onorm, inorm, h4, f4 = _tc_prologue(nf, degp4, W_in, b_in.reshape(1, H))

    for i in range(NL):
        s4 = _sc_aggregate(f4, srcp, dstp, zblk)
        z4, stats = _tc_mm(s4, inorm, b_conv[i].reshape(1, H), W_conv[i])
        var2 = _tc_stats2(z4, stats)
        is_res = (i % 2 == 1)
        has_next = (i < NL - 1)
        outs = _tc_layer(z4, stats, var2,
                         gamma[i].reshape(1, H), beta[i].reshape(1, H),
                         h4, onorm, is_res, has_next)
        if is_res and has_next:
            h4, f4 = outs
        elif is_res:
            h4 = outs[0] if isinstance(outs, (list, tuple)) else outs
        else:
            f4 = outs[0] if isinstance(outs, (list, tuple)) else outs

    wp = jnp.pad(W_pose, ((0, 0), (0, 128 - W_pose.shape[1])))
    bp = jnp.pad(b_pose, (0, 128 - b_pose.shape[0])).reshape(1, 128)
    wc = jnp.pad(W_cls, ((0, 0), (0, 128 - W_cls.shape[1])))
    bc = jnp.pad(b_cls, (0, 128 - b_cls.shape[0])).reshape(1, 128)
    pose_pad, label_pad = _tc_epilogue(h4, wp, bp, wc, bc)
    pose = pose_pad[:N, :W_pose.shape[1]]
    label = label_pad[0:1, :W_cls.shape[1]]
    return (pose, label)
